# Initial kernel scaffold; baseline (speedup 1.0000x reference)
import jax
import jax.numpy as jnp
from jax.experimental import pallas as pl


def kernel(token_idx, weight):
    raise NotImplementedError("write your pallas kernel here")


# speedup vs baseline: 1.1880x; 1.1880x over previous
import functools

import jax
import jax.numpy as jnp
from jax import lax
from jax.experimental import pallas as pl
from jax.experimental.pallas import tpu as pltpu
from jax.experimental.pallas import tpu_sc as plsc

NUM_EMB = 1000000
D = 32
NC = 2
NS = 16
NW = NC * NS
B = 16384 * 50
CHUNK = 128
NCHUNK = B // (NW * CHUNK)


def _make_kernel():
  mesh = plsc.VectorSubcoreMesh(core_axis_name="c", subcore_axis_name="s")

  @functools.partial(
      pl.kernel,
      out_type=jax.ShapeDtypeStruct((NW, NCHUNK, CHUNK, D), jnp.float32),
      mesh=mesh,
      compiler_params=pltpu.CompilerParams(use_tc_tiling_on_sc=False),
      scratch_types=[
          pltpu.VMEM((NCHUNK, CHUNK), jnp.int32),
          pltpu.VMEM((CHUNK, D), jnp.float32),
          pltpu.SemaphoreType.DMA,
      ],
  )
  def emb_kernel(idx_hbm, table_hbm, out_hbm, idx_v, rows_v, sem):
    wid = lax.axis_index("s") * NC + lax.axis_index("c")
    pltpu.sync_copy(idx_hbm.at[wid], idx_v)

    def step(g, carry):
      pltpu.async_copy(table_hbm.at[idx_v.at[g]], rows_v, sem).wait()
      pltpu.sync_copy(rows_v, out_hbm.at[wid, g])
      return carry

    lax.fori_loop(0, NCHUNK, step, 0)

  return emb_kernel


_emb = _make_kernel()


@jax.jit
def kernel(token_idx, weight):
  idx = token_idx.reshape(NW, NCHUNK, CHUNK).astype(jnp.int32)
  out = _emb(idx, weight)
  return out.reshape(16384, 50, D)


# speedup vs baseline: 1.2933x; 1.0886x over previous
import functools

import jax
import jax.numpy as jnp
from jax import lax
from jax.experimental import pallas as pl
from jax.experimental.pallas import tpu as pltpu
from jax.experimental.pallas import tpu_sc as plsc

NUM_EMB = 1000000
D = 32
NC = 2
NS = 16
NW = NC * NS
B = 16384 * 50
CHUNK = 128
NCHUNK = B // (NW * CHUNK)
K = 4
SUPER = K * CHUNK
NSUPER = NCHUNK // K
PERW = B // NW


def _make_kernel():
  mesh = plsc.VectorSubcoreMesh(core_axis_name="c", subcore_axis_name="s")

  @functools.partial(
      pl.kernel,
      out_type=jax.ShapeDtypeStruct((NW, PERW, D), jnp.float32),
      mesh=mesh,
      compiler_params=pltpu.CompilerParams(use_tc_tiling_on_sc=False),
      scratch_types=[
          pltpu.VMEM((NCHUNK, CHUNK), jnp.int32),
          [pltpu.VMEM((SUPER, D), jnp.float32) for _ in range(2)],
          [pltpu.SemaphoreType.DMA for _ in range(2)],
          [pltpu.SemaphoreType.DMA for _ in range(2)],
      ],
  )
  def emb_kernel(idx_hbm, table_hbm, out_hbm, idx_v, bufs, gsems, osems):
    wid = lax.axis_index("s") * NC + lax.axis_index("c")
    pltpu.sync_copy(idx_hbm.at[wid], idx_v)

    def fire_gathers(s, b):
      for j in range(K):
        pltpu.async_copy(
            table_hbm.at[idx_v.at[s * K + j]],
            bufs[b].at[pl.ds(j * CHUNK, CHUNK)],
            gsems[b],
        )

    def drain_gathers(b):
      pltpu.make_async_copy(
          table_hbm.at[pl.ds(0, SUPER)], bufs[b], gsems[b]
      ).wait()

    def start_out(s, b):
      pltpu.async_copy(
          bufs[b], out_hbm.at[wid, pl.ds(s * SUPER, SUPER)], osems[b]
      )

    def drain_out(b):
      pltpu.make_async_copy(
          bufs[b], out_hbm.at[wid, pl.ds(0, SUPER)], osems[b]
      ).wait()

    fire_gathers(0, 0)

    def body(g, carry):
      for b in range(2):
        t = g * 2 + b
        nb = 1 - b
        drain_gathers(b)

        @pl.when(t + 1 < NSUPER)
        def _prefetch():
          @pl.when(t >= 1)
          def _():
            drain_out(nb)

          fire_gathers(t + 1, nb)

        start_out(t, b)
      return carry

    lax.fori_loop(0, NSUPER // 2, body, 0)
    drain_out((NSUPER - 1) % 2)

  return emb_kernel


_emb = _make_kernel()


@jax.jit
def kernel(token_idx, weight):
  idx = token_idx.reshape(NW, NCHUNK, CHUNK).astype(jnp.int32)
  out = _emb(idx, weight)
  return out.reshape(16384, 50, D)


# speedup vs baseline: 1.7925x; 1.3860x over previous
import functools

import jax
import jax.numpy as jnp
from jax import lax
from jax.experimental import pallas as pl
from jax.experimental.pallas import tpu as pltpu
from jax.experimental.pallas import tpu_sc as plsc

NUM_EMB = 1000000
D = 32
T = 50
NROW = 16384
NC = 2
NS = 16
NW = NC * NS
RPW = NROW // NW
R = 16
NSUPER = RPW // R


def _make_kernel():
  mesh = plsc.VectorSubcoreMesh(core_axis_name="c", subcore_axis_name="s")

  @functools.partial(
      pl.kernel,
      out_type=jax.ShapeDtypeStruct((NROW, T, D), jnp.float32),
      mesh=mesh,
      compiler_params=pltpu.CompilerParams(use_tc_tiling_on_sc=False),
      scratch_types=[
          pltpu.VMEM((RPW, T), jnp.int32),
          [pltpu.VMEM((R, T, D), jnp.float32) for _ in range(2)],
          [pltpu.SemaphoreType.DMA for _ in range(2)],
          [pltpu.SemaphoreType.DMA for _ in range(2)],
      ],
  )
  def emb_kernel(idx_hbm, table_hbm, out_hbm, idx_v, bufs, gsems, osems):
    wid = lax.axis_index("s") * NC + lax.axis_index("c")
    row0 = wid * RPW
    pltpu.sync_copy(idx_hbm.at[pl.ds(row0, RPW)], idx_v)

    def fire_gather(s, b):
      for j in range(R):
        pltpu.async_copy(
            table_hbm.at[idx_v.at[s * R + j]],
            bufs[b].at[j],
            gsems[b],
        )

    def drain_gather(b):
      pltpu.make_async_copy(
          out_hbm.at[pl.ds(0, R)], bufs[b], gsems[b]
      ).wait()

    def start_out(s, b):
      pltpu.async_copy(
          bufs[b], out_hbm.at[pl.ds(row0 + s * R, R)], osems[b]
      )

    def drain_out(b):
      pltpu.make_async_copy(
          bufs[b], out_hbm.at[pl.ds(0, R)], osems[b]
      ).wait()

    fire_gather(0, 0)

    def body(g, carry):
      for b in range(2):
        t = g * 2 + b
        nb = 1 - b
        drain_gather(b)

        @pl.when(t + 1 < NSUPER)
        def _prefetch():
          @pl.when(t >= 1)
          def _():
            drain_out(nb)

          fire_gather(t + 1, nb)

        start_out(t, b)
      return carry

    lax.fori_loop(0, NSUPER // 2, body, 0)
    drain_out((NSUPER - 1) % 2)

  return emb_kernel


_emb = _make_kernel()


@jax.jit
def kernel(token_idx, weight):
  return _emb(token_idx, weight)
